# baseline (speedup 1.0000x reference)
import functools

import jax
import jax.numpy as jnp
from jax import lax
from jax.experimental import pallas as pl
from jax.experimental.pallas import tpu as pltpu
from jax.experimental.pallas import tpu_sc as plsc

N = 10000
E = 160000
D = 256
H = 4
DH = 64
EOUT = 16

NC = 2
NS = 16
NW = NC * NS
L = 16

CA = 32
NCA = E // CA
CC = 64
NCC = E // CC
NPAD = 10240
NPT = NPAD // NS

_f32 = jnp.float32
_i32 = jnp.int32

_mesh = plsc.VectorSubcoreMesh(core_axis_name="c", subcore_axis_name="s")
_sc_params = pltpu.CompilerParams(needs_layout_passes=False)


def _qkv_body(xs_ref, xd_ref, wk_ref, wq_ref, wv_ref,
              klo, khi, qlo, qhi, vlo, vhi):
    k = jnp.dot(xs_ref[...], wk_ref[...], preferred_element_type=_f32)
    q = jnp.dot(xd_ref[...], wq_ref[...], preferred_element_type=_f32)
    v = jnp.dot(xs_ref[...], wv_ref[...], preferred_element_type=_f32)
    klo[...] = k[:, :128]
    khi[...] = k[:, 128:]
    qlo[...] = q[:, :128]
    qhi[...] = q[:, 128:]
    vlo[...] = v[:, :128]
    vhi[...] = v[:, 128:]


def _qkv(x_src, x_dst, Wk, Wq, Wv):
    BN = 2000
    grid = (N // BN,)
    half = jax.ShapeDtypeStruct((N, 128), _f32)
    bx = pl.BlockSpec((BN, D), lambda i: (i, 0))
    bw = pl.BlockSpec((D, D), lambda i: (0, 0))
    bo = pl.BlockSpec((BN, 128), lambda i: (i, 0))
    return pl.pallas_call(
        _qkv_body,
        grid=grid,
        in_specs=[bx, bx, bw, bw, bw],
        out_specs=[bo] * 6,
        out_shape=[half] * 6,
    )(x_src, x_dst, Wk, Wq, Wv)


def _edense_body(ea_ref, we_ref, elo, ehi):
    e = jnp.dot(ea_ref[...], we_ref[...], preferred_element_type=_f32)
    elo[...] = e[:, :128]
    ehi[...] = e[:, 128:]


def _edense(ea, We):
    BE = 8000
    c = ea.shape[1]
    grid = (E // BE,)
    half = jax.ShapeDtypeStruct((E, 128), _f32)
    return pl.pallas_call(
        _edense_body,
        grid=grid,
        in_specs=[pl.BlockSpec((BE, c), lambda i: (i, 0)),
                  pl.BlockSpec((c, D), lambda i: (0, 0))],
        out_specs=[pl.BlockSpec((BE, 128), lambda i: (i, 0))] * 2,
        out_shape=[half] * 2,
    )(ea, We)


def _recip_body(s_ref, r_ref):
    tot = jnp.sum(s_ref[...], axis=0, keepdims=True)
    r_ref[...] = 1.0 / (tot + 1e-16)


def _recip(s_parts):
    flat = s_parts.reshape(NW, H * NPAD)
    r = pl.pallas_call(
        _recip_body,
        out_shape=jax.ShapeDtypeStruct((1, H * NPAD), _f32),
    )(flat)
    return r.reshape(H * NPAD)


def _bn_body(alo_ref, ahi_ref, x_ref, g_ref, b_ref, o_ref):
    al = alo_ref[:N] + alo_ref[NPAD:NPAD + N]
    ah = ahi_ref[:N] + ahi_ref[NPAD:NPAD + N]
    x = jnp.concatenate([al, ah], axis=1) + x_ref[...]
    n = x.shape[0]
    mu = jnp.sum(x, axis=0, keepdims=True) / n
    xc = x - mu
    var = jnp.sum(xc * xc, axis=0, keepdims=True) / n
    y = xc * lax.rsqrt(var + 1e-5) * g_ref[...] + b_ref[...]
    o_ref[...] = jnp.where(y > 0, y, jnp.exp(y) - 1.0)


def _bn_elu(agg_lo, agg_hi, x, g, b):
    return pl.pallas_call(
        _bn_body,
        out_shape=jax.ShapeDtypeStruct((N, D), _f32),
    )(agg_lo, agg_hi, x, g.reshape(1, D), b.reshape(1, D))


def _eout_bn_elu_body(eaT_ref, We_ref, Weo_ref, g_ref, b_ref, o_ref):
    M = jnp.dot(We_ref[...], Weo_ref[...], preferred_element_type=_f32)
    z = jnp.dot(M.T, eaT_ref[...], preferred_element_type=_f32)
    n = z.shape[1]
    mu = jnp.sum(z, axis=1, keepdims=True) / n
    zc = z - mu
    var = jnp.sum(zc * zc, axis=1, keepdims=True) / n
    y = zc * lax.rsqrt(var + 1e-5) * g_ref[...] + b_ref[...]
    o_ref[...] = jnp.where(y > 0, y, jnp.exp(y) - 1.0)


def _eout_bn_elu(ea, We, Weo, g, b):
    zT = pl.pallas_call(
        _eout_bn_elu_body,
        out_shape=jax.ShapeDtypeStruct((EOUT, ea.shape[0]), _f32),
    )(ea.T, We, Weo, g.reshape(EOUT, 1), b.reshape(EOUT, 1))
    return zT.T


@functools.partial(
    pl.kernel,
    out_type=(jax.ShapeDtypeStruct((H * E,), _f32),
              jax.ShapeDtypeStruct((NW * H * NPAD,), _f32)),
    mesh=_mesh,
    compiler_params=_sc_params,
    scratch_types=[
        [pltpu.VMEM((CA,), _i32)] * 2,
        [pltpu.VMEM((CA,), _i32)] * 2,
        [pltpu.VMEM((CA, 128), _f32)] * 2,
        [pltpu.VMEM((CA, 128), _f32)] * 2,
        [pltpu.VMEM((CA, 128), _f32)] * 2,
        [pltpu.VMEM((CA, 128), _f32)] * 2,
        [pltpu.VMEM((CA, 128), _f32)] * 2,
        [pltpu.VMEM((CA, 128), _f32)] * 2,
        [pltpu.VMEM((CA,), _f32)] * H,
        pltpu.VMEM((H * NPAD,), _f32),
        pltpu.SemaphoreType.DMA,
    ],
)
def _sc_pass_a(klo, khi, qlo, qhi, elo, ehi, srcI, dstI,
               exT_out, s_out,
               srcv2, dstv2, bkl2, bkh2, bql2, bqh2, bel2, beh2, exrows,
               s_loc, sem):
    cid = lax.axis_index("c")
    sid = lax.axis_index("s")
    wid = cid * NS + sid
    iota = lax.iota(_i32, 16)
    zero16 = jnp.zeros((16,), _f32)

    def zs(i, carry):
        s_loc[pl.ds(i * 16, 16)] = zero16
        return carry

    lax.fori_loop(0, H * NPAD // 16, zs, 0, unroll=8)

    trips = (NCA + NW - 1) // NW
    tabs = (bkl2, bkh2, bql2, bqh2, bel2, beh2)

    def prefetch(g, par):
        @pl.when(g < NCA)
        def _():
            base = g * CA
            pltpu.sync_copy(srcI.at[pl.ds(base, CA)], srcv2[par])
            pltpu.sync_copy(dstI.at[pl.ds(base, CA)], dstv2[par])
            pltpu.async_copy(klo.at[srcv2[par]], bkl2[par], sem)
            pltpu.async_copy(khi.at[srcv2[par]], bkh2[par], sem)
            pltpu.async_copy(qlo.at[dstv2[par]], bql2[par], sem)
            pltpu.async_copy(qhi.at[dstv2[par]], bqh2[par], sem)
            pltpu.async_copy(elo.at[pl.ds(base, CA)], bel2[par], sem)
            pltpu.async_copy(ehi.at[pl.ds(base, CA)], beh2[par], sem)

    def subchunk(g, par):
        bkl, bkh, bql, bqh, bel, beh = (t[par] for t in tabs)
        srcv, dstv = srcv2[par], dstv2[par]

        @pl.when(g < NCA)
        def _():
            base = g * CA
            pltpu.make_async_copy(klo.at[srcv], bkl, sem).wait()
            pltpu.make_async_copy(khi.at[srcv], bkh, sem).wait()
            pltpu.make_async_copy(qlo.at[dstv], bql, sem).wait()
            pltpu.make_async_copy(qhi.at[dstv], bqh, sem).wait()
            pltpu.make_async_copy(elo.at[pl.ds(base, CA)], bel, sem).wait()
            pltpu.make_async_copy(ehi.at[pl.ds(base, CA)], beh, sem).wait()
            prefetch(g + NW, 1 - par)
            for eg in range(CA // 16):
                sl = pl.ds(eg * 16, 16)
                dvv = dstv[sl]

                def edge(j, lgs):
                    i = eg * 16 + j
                    lane = iota == j
                    out = []
                    for h in range(H):
                        qb = bql if h < 2 else bqh
                        kb = bkl if h < 2 else bkh
                        eb = bel if h < 2 else beh
                        w = zero16
                        for c in range(4):
                            dsl = pl.ds(((h % 2) * 4 + c) * 16, 16)
                            w = w + qb[i, dsl] * (kb[i, dsl] + eb[i, dsl])
                        tot = plsc.cumsum(w)[15]
                        out.append(jnp.where(lane, tot, lgs[h]))
                    return tuple(out)

                lgs = lax.fori_loop(0, 16, edge,
                                    (zero16,) * H, unroll=8)
                for h in range(H):
                    ex = jnp.exp(lgs[h] * 0.125)
                    exrows[h][sl] = ex
                    plsc.addupdate_scatter(s_loc, [dvv + h * NPAD], ex)
            for h in range(H):
                pltpu.sync_copy(exrows[h], exT_out.at[pl.ds(h * E + base, CA)])

    prefetch(wid, 0)
    trips2 = (trips + 1) // 2

    def chunk2(t2, carry):
        g = wid + (2 * t2) * NW
        subchunk(g, 0)
        subchunk(g + NW, 1)
        return carry

    lax.fori_loop(0, trips2, chunk2, 0)
    pltpu.sync_copy(s_loc, s_out.at[pl.ds(wid * H * NPAD, H * NPAD)])


CB = 128
NCB = E // CB


@functools.partial(
    pl.kernel,
    out_type=jax.ShapeDtypeStruct((H * E,), _f32),
    mesh=_mesh,
    compiler_params=_sc_params,
    scratch_types=[
        pltpu.VMEM((CB,), _i32),
        [pltpu.VMEM((CB,), _f32)] * H,
        [pltpu.VMEM((CB,), _f32)] * H,
        pltpu.VMEM((H * NPAD,), _f32),
        pltpu.SemaphoreType.DMA,
    ],
)
def _sc_pass_b(exT, rI, dstI, al_out, dstv, exv, av, r_loc, sem):
    cid = lax.axis_index("c")
    sid = lax.axis_index("s")
    wid = cid * NS + sid

    pltpu.sync_copy(rI, r_loc)
    trips = (NCB + NW - 1) // NW

    def chunk(t, carry):
        g = wid + t * NW

        @pl.when(g < NCB)
        def _():
            base = g * CB
            pltpu.sync_copy(dstI.at[pl.ds(base, CB)], dstv)
            for h in range(H):
                pltpu.sync_copy(exT.at[pl.ds(h * E + base, CB)], exv[h])
            for eg in range(CB // 16):
                sl = pl.ds(eg * 16, 16)
                dvv = dstv[sl]
                for h in range(H):
                    rv = plsc.load_gather(r_loc, [dvv + h * NPAD])
                    av[h][sl] = exv[h][sl] * rv
            for h in range(H):
                pltpu.sync_copy(av[h], al_out.at[pl.ds(h * E + base, CB)])

        return carry

    lax.fori_loop(0, trips, chunk, 0)


def _make_pass_c(half):
    h0 = 2 * half

    @functools.partial(
        pl.kernel,
        out_type=jax.ShapeDtypeStruct((NC * NPAD, 128), _f32),
        mesh=_mesh,
        compiler_params=_sc_params,
        scratch_types=[
            [pltpu.VMEM((CC,), _i32)] * 2,
            [pltpu.VMEM((CC,), _i32)] * 2,
            [pltpu.VMEM((CC, 128), _f32)] * 2,
            [pltpu.VMEM((CC, 128), _f32)] * 2,
            [[pltpu.VMEM((CC,), _f32)] * 2] * 2,
            pltpu.VMEM_SHARED((NPAD, 128), _f32),
            pltpu.SemaphoreType.DMA,
        ],
    )
    def body(vh, eh, alT, srcI, dstI, zeros128,
             agg_out,
             srcv2, dstv2, vrows2, erows2, alb2, acc_sh, sem):
        cid = lax.axis_index("c")
        sid = lax.axis_index("s")
        wid = cid * NS + sid

        pltpu.sync_copy(zeros128.at[pl.ds(sid * NPT, NPT)],
                        acc_sh.at[pl.ds(sid * NPT, NPT)])
        plsc.subcore_barrier()

        trips = (NCC + NW - 1) // NW

        def prefetch(g, par):
            @pl.when(g < NCC)
            def _():
                base = g * CC
                pltpu.sync_copy(srcI.at[pl.ds(base, CC)], srcv2[par])
                pltpu.sync_copy(dstI.at[pl.ds(base, CC)], dstv2[par])
                pltpu.async_copy(vh.at[srcv2[par]], vrows2[par], sem)
                pltpu.async_copy(eh.at[pl.ds(base, CC)], erows2[par], sem)
                pltpu.async_copy(alT.at[pl.ds(h0 * E + base, CC)],
                                 alb2[par][0], sem)
                pltpu.async_copy(alT.at[pl.ds((h0 + 1) * E + base, CC)],
                                 alb2[par][1], sem)

        def subchunk(g, par):
            srcv, dstv = srcv2[par], dstv2[par]
            vrows, erows, alb = vrows2[par], erows2[par], alb2[par]

            @pl.when(g < NCC)
            def _():
                base = g * CC
                pltpu.make_async_copy(vh.at[srcv], vrows, sem).wait()
                pltpu.make_async_copy(eh.at[pl.ds(base, CC)], erows,
                                      sem).wait()
                pltpu.make_async_copy(alT.at[pl.ds(h0 * E + base, CC)],
                                      alb[0], sem).wait()
                pltpu.make_async_copy(alT.at[pl.ds((h0 + 1) * E + base, CC)],
                                      alb[1], sem).wait()
                prefetch(g + NW, 1 - par)
                for eg in range(CC // 16):
                    sl = pl.ds(eg * 16, 16)
                    av = [alb[0][sl], alb[1][sl]]
                    for j in range(16):
                        i = eg * 16 + j
                        s0 = jnp.full((16,), av[0][j], _f32)
                        s1 = jnp.full((16,), av[1][j], _f32)
                        for c in range(8):
                            sc = pl.ds(c * 16, 16)
                            s = s0 if c < 4 else s1
                            vrows[i, sc] = (vrows[i, sc] + erows[i, sc]) * s
                pltpu.sync_copy(vrows, acc_sh.at[dstv], add=True)

        prefetch(wid, 0)
        trips2 = (trips + 1) // 2

        def chunk2(t2, carry):
            g = wid + (2 * t2) * NW
            subchunk(g, 0)
            subchunk(g + NW, 1)
            return carry

        lax.fori_loop(0, trips2, chunk2, 0)
        plsc.subcore_barrier()
        pltpu.sync_copy(acc_sh.at[pl.ds(sid * NPT, NPT)],
                        agg_out.at[pl.ds(cid * NPAD + sid * NPT, NPT)])

    return body


_sc_pass_c = (_make_pass_c(0), _make_pass_c(1))


def _rel_conv(x_src, x_dst, ei, ea, Wk, Wq, Wv, We, zeros128):
    src = ei[0]
    dst = ei[1]
    klo, khi, qlo, qhi, vlo, vhi = _qkv(x_src, x_dst, Wk, Wq, Wv)
    elo, ehi = _edense(ea, We)
    exT, s_parts = _sc_pass_a(klo, khi, qlo, qhi, elo, ehi, src, dst)
    r = _recip(s_parts)
    alT = _sc_pass_b(exT, r, dst)
    agg_lo = _sc_pass_c[0](vlo, elo, alT, src, dst, zeros128)
    agg_hi = _sc_pass_c[1](vhi, ehi, alT, src, dst, zeros128)
    return agg_lo, agg_hi


def kernel(x_user, x_item, ei_ut, ei_tu, ea_ut, ea_tu,
           Wk_ut, Wq_ut, Wv_ut, We_ut, Weo_ut,
           Wk_tu, Wq_tu, Wv_tu, We_tu, Weo_tu,
           g_user, b_user, g_item, b_item,
           ge_ut, be_ut, ge_tu, be_tu):
    zeros128 = jnp.zeros((NPAD, 128), _f32)
    ilo, ihi = _rel_conv(x_user, x_item, ei_ut, ea_ut,
                         Wk_ut, Wq_ut, Wv_ut, We_ut, zeros128)
    ulo, uhi = _rel_conv(x_item, x_user, ei_tu, ea_tu,
                         Wk_tu, Wq_tu, Wv_tu, We_tu, zeros128)
    xu = _bn_elu(ulo, uhi, x_user, g_user, b_user)
    xi = _bn_elu(ilo, ihi, x_item, g_item, b_item)
    eu = _eout_bn_elu(ea_ut, We_ut, Weo_ut, ge_ut, be_ut)
    et = _eout_bn_elu(ea_tu, We_tu, Weo_tu, ge_tu, be_tu)
    return (xu, xi, eu, et)


# speedup vs baseline: 1.3051x; 1.3051x over previous
import functools

import jax
import jax.numpy as jnp
from jax import lax
from jax.experimental import pallas as pl
from jax.experimental.pallas import tpu as pltpu
from jax.experimental.pallas import tpu_sc as plsc

N = 10000
E = 160000
D = 256
H = 4
DH = 64
EOUT = 16

NC = 2
NS = 16
NW = NC * NS
L = 16

CA = 32
NCA = E // CA
CC = 64
NCC = E // CC
NPAD = 10240
NPT = NPAD // NS

_f32 = jnp.float32
_i32 = jnp.int32

_mesh = plsc.VectorSubcoreMesh(core_axis_name="c", subcore_axis_name="s")
_sc_params = pltpu.CompilerParams(needs_layout_passes=False)


def _qkv_body(xs_ref, xd_ref, wk_ref, wq_ref, wv_ref,
              klo, khi, qlo, qhi, vlo, vhi):
    k = jnp.dot(xs_ref[...], wk_ref[...], preferred_element_type=_f32)
    q = jnp.dot(xd_ref[...], wq_ref[...], preferred_element_type=_f32)
    v = jnp.dot(xs_ref[...], wv_ref[...], preferred_element_type=_f32)
    klo[...] = k[:, :128]
    khi[...] = k[:, 128:]
    qlo[...] = q[:, :128]
    qhi[...] = q[:, 128:]
    vlo[...] = v[:, :128]
    vhi[...] = v[:, 128:]


def _qkv(x_src, x_dst, Wk, Wq, Wv):
    BN = 2000
    grid = (N // BN,)
    half = jax.ShapeDtypeStruct((N, 128), _f32)
    bx = pl.BlockSpec((BN, D), lambda i: (i, 0))
    bw = pl.BlockSpec((D, D), lambda i: (0, 0))
    bo = pl.BlockSpec((BN, 128), lambda i: (i, 0))
    return pl.pallas_call(
        _qkv_body,
        grid=grid,
        in_specs=[bx, bx, bw, bw, bw],
        out_specs=[bo] * 6,
        out_shape=[half] * 6,
    )(x_src, x_dst, Wk, Wq, Wv)


def _edense_body(ea_ref, we_ref, elo, ehi):
    e = jnp.dot(ea_ref[...], we_ref[...], preferred_element_type=_f32)
    elo[...] = e[:, :128]
    ehi[...] = e[:, 128:]


def _edense(ea, We):
    BE = 8000
    c = ea.shape[1]
    grid = (E // BE,)
    half = jax.ShapeDtypeStruct((E, 128), _f32)
    return pl.pallas_call(
        _edense_body,
        grid=grid,
        in_specs=[pl.BlockSpec((BE, c), lambda i: (i, 0)),
                  pl.BlockSpec((c, D), lambda i: (0, 0))],
        out_specs=[pl.BlockSpec((BE, 128), lambda i: (i, 0))] * 2,
        out_shape=[half] * 2,
    )(ea, We)


def _recip_body(s_ref, r_ref):
    tot = jnp.sum(s_ref[...], axis=0, keepdims=True)
    r_ref[...] = 1.0 / (tot + 1e-16)


def _recip(s_parts):
    flat = s_parts.reshape(NW, H * NPAD)
    r = pl.pallas_call(
        _recip_body,
        out_shape=jax.ShapeDtypeStruct((1, H * NPAD), _f32),
    )(flat)
    return r.reshape(H * NPAD)


def _bn_body(alo_ref, ahi_ref, x_ref, g_ref, b_ref, o_ref):
    al = alo_ref[:N] + alo_ref[NPAD:NPAD + N]
    ah = ahi_ref[:N] + ahi_ref[NPAD:NPAD + N]
    x = jnp.concatenate([al, ah], axis=1) + x_ref[...]
    n = x.shape[0]
    mu = jnp.sum(x, axis=0, keepdims=True) / n
    xc = x - mu
    var = jnp.sum(xc * xc, axis=0, keepdims=True) / n
    y = xc * lax.rsqrt(var + 1e-5) * g_ref[...] + b_ref[...]
    o_ref[...] = jnp.where(y > 0, y, jnp.exp(y) - 1.0)


def _bn_elu(agg_lo, agg_hi, x, g, b):
    return pl.pallas_call(
        _bn_body,
        out_shape=jax.ShapeDtypeStruct((N, D), _f32),
    )(agg_lo, agg_hi, x, g.reshape(1, D), b.reshape(1, D))


def _eout_bn_elu_body(eaT_ref, We_ref, Weo_ref, g_ref, b_ref, o_ref):
    M = jnp.dot(We_ref[...], Weo_ref[...], preferred_element_type=_f32)
    z = jnp.dot(M.T, eaT_ref[...], preferred_element_type=_f32)
    n = z.shape[1]
    mu = jnp.sum(z, axis=1, keepdims=True) / n
    zc = z - mu
    var = jnp.sum(zc * zc, axis=1, keepdims=True) / n
    y = zc * lax.rsqrt(var + 1e-5) * g_ref[...] + b_ref[...]
    o_ref[...] = jnp.where(y > 0, y, jnp.exp(y) - 1.0)


def _eout_bn_elu(ea, We, Weo, g, b):
    zT = pl.pallas_call(
        _eout_bn_elu_body,
        out_shape=jax.ShapeDtypeStruct((EOUT, ea.shape[0]), _f32),
    )(ea.T, We, Weo, g.reshape(EOUT, 1), b.reshape(EOUT, 1))
    return zT.T


@functools.partial(
    pl.kernel,
    out_type=(jax.ShapeDtypeStruct((H * E,), _f32),
              jax.ShapeDtypeStruct((NW * H * NPAD,), _f32)),
    mesh=_mesh,
    compiler_params=_sc_params,
    scratch_types=[
        [pltpu.VMEM((CA,), _i32)] * 2,
        [pltpu.VMEM((CA,), _i32)] * 2,
        [pltpu.VMEM((CA, 128), _f32)] * 2,
        [pltpu.VMEM((CA, 128), _f32)] * 2,
        [pltpu.VMEM((CA, 128), _f32)] * 2,
        [pltpu.VMEM((CA, 128), _f32)] * 2,
        [pltpu.VMEM((CA, 128), _f32)] * 2,
        [pltpu.VMEM((CA, 128), _f32)] * 2,
        [pltpu.VMEM((CA,), _f32)] * H,
        pltpu.VMEM((H * NPAD,), _f32),
        pltpu.SemaphoreType.DMA,
    ],
)
def _sc_pass_a(klo, khi, qlo, qhi, elo, ehi, srcI, dstI,
               exT_out, s_out,
               srcv2, dstv2, bkl2, bkh2, bql2, bqh2, bel2, beh2, exrows,
               s_loc, sem):
    cid = lax.axis_index("c")
    sid = lax.axis_index("s")
    wid = cid * NS + sid
    iota = lax.iota(_i32, 16)
    zero16 = jnp.zeros((16,), _f32)

    def zs(i, carry):
        s_loc[pl.ds(i * 16, 16)] = zero16
        return carry

    lax.fori_loop(0, H * NPAD // 16, zs, 0, unroll=8)

    trips = (NCA + NW - 1) // NW
    tabs = (bkl2, bkh2, bql2, bqh2, bel2, beh2)

    def prefetch(g, par):
        @pl.when(g < NCA)
        def _():
            base = g * CA
            pltpu.sync_copy(srcI.at[pl.ds(base, CA)], srcv2[par])
            pltpu.sync_copy(dstI.at[pl.ds(base, CA)], dstv2[par])
            pltpu.async_copy(klo.at[srcv2[par]], bkl2[par], sem)
            pltpu.async_copy(khi.at[srcv2[par]], bkh2[par], sem)
            pltpu.async_copy(qlo.at[dstv2[par]], bql2[par], sem)
            pltpu.async_copy(qhi.at[dstv2[par]], bqh2[par], sem)
            pltpu.async_copy(elo.at[pl.ds(base, CA)], bel2[par], sem)
            pltpu.async_copy(ehi.at[pl.ds(base, CA)], beh2[par], sem)

    def subchunk(g, par):
        bkl, bkh, bql, bqh, bel, beh = (t[par] for t in tabs)
        srcv, dstv = srcv2[par], dstv2[par]

        @pl.when(g < NCA)
        def _():
            base = g * CA
            pltpu.make_async_copy(klo.at[srcv], bkl, sem).wait()
            pltpu.make_async_copy(khi.at[srcv], bkh, sem).wait()
            pltpu.make_async_copy(qlo.at[dstv], bql, sem).wait()
            pltpu.make_async_copy(qhi.at[dstv], bqh, sem).wait()
            pltpu.make_async_copy(elo.at[pl.ds(base, CA)], bel, sem).wait()
            pltpu.make_async_copy(ehi.at[pl.ds(base, CA)], beh, sem).wait()
            prefetch(g + NW, 1 - par)
            for eg in range(CA // 16):
                sl = pl.ds(eg * 16, 16)
                dvv = dstv[sl]

                def edge(j, lgs):
                    i = eg * 16 + j
                    lane = iota == j
                    out = []
                    for h in range(H):
                        qb = bql if h < 2 else bqh
                        kb = bkl if h < 2 else bkh
                        eb = bel if h < 2 else beh
                        w = zero16
                        for c in range(4):
                            dsl = pl.ds(((h % 2) * 4 + c) * 16, 16)
                            w = w + qb[i, dsl] * (kb[i, dsl] + eb[i, dsl])
                        tot = plsc.cumsum(w)[15]
                        out.append(jnp.where(lane, tot, lgs[h]))
                    return tuple(out)

                lgs = lax.fori_loop(0, 16, edge,
                                    (zero16,) * H, unroll=2)
                for h in range(H):
                    ex = jnp.exp(lgs[h] * 0.125)
                    exrows[h][sl] = ex
                    plsc.addupdate_scatter(s_loc, [dvv + h * NPAD], ex)
            for h in range(H):
                pltpu.sync_copy(exrows[h], exT_out.at[pl.ds(h * E + base, CA)])

    prefetch(wid, 0)
    trips2 = (trips + 1) // 2

    def chunk2(t2, carry):
        g = wid + (2 * t2) * NW
        subchunk(g, 0)
        subchunk(g + NW, 1)
        return carry

    lax.fori_loop(0, trips2, chunk2, 0)
    pltpu.sync_copy(s_loc, s_out.at[pl.ds(wid * H * NPAD, H * NPAD)])


CB = 128
NCB = E // CB


@functools.partial(
    pl.kernel,
    out_type=jax.ShapeDtypeStruct((H * E,), _f32),
    mesh=_mesh,
    compiler_params=_sc_params,
    scratch_types=[
        pltpu.VMEM((CB,), _i32),
        [pltpu.VMEM((CB,), _f32)] * H,
        [pltpu.VMEM((CB,), _f32)] * H,
        pltpu.VMEM((H * NPAD,), _f32),
        pltpu.SemaphoreType.DMA,
    ],
)
def _sc_pass_b(exT, rI, dstI, al_out, dstv, exv, av, r_loc, sem):
    cid = lax.axis_index("c")
    sid = lax.axis_index("s")
    wid = cid * NS + sid

    pltpu.sync_copy(rI, r_loc)
    trips = (NCB + NW - 1) // NW

    def chunk(t, carry):
        g = wid + t * NW

        @pl.when(g < NCB)
        def _():
            base = g * CB
            pltpu.sync_copy(dstI.at[pl.ds(base, CB)], dstv)
            for h in range(H):
                pltpu.sync_copy(exT.at[pl.ds(h * E + base, CB)], exv[h])
            for eg in range(CB // 16):
                sl = pl.ds(eg * 16, 16)
                dvv = dstv[sl]
                for h in range(H):
                    rv = plsc.load_gather(r_loc, [dvv + h * NPAD])
                    av[h][sl] = exv[h][sl] * rv
            for h in range(H):
                pltpu.sync_copy(av[h], al_out.at[pl.ds(h * E + base, CB)])

        return carry

    lax.fori_loop(0, trips, chunk, 0)


def _make_pass_c(half):
    h0 = 2 * half

    @functools.partial(
        pl.kernel,
        out_type=jax.ShapeDtypeStruct((NC * NPAD, 128), _f32),
        mesh=_mesh,
        compiler_params=_sc_params,
        scratch_types=[
            [pltpu.VMEM((CC,), _i32)] * 2,
            [pltpu.VMEM((CC,), _i32)] * 2,
            [pltpu.VMEM((CC, 128), _f32)] * 2,
            [pltpu.VMEM((CC, 128), _f32)] * 2,
            [[pltpu.VMEM((CC,), _f32)] * 2] * 2,
            pltpu.VMEM_SHARED((NPAD, 128), _f32),
            pltpu.SemaphoreType.DMA,
        ],
    )
    def body(vh, eh, alT, srcI, dstI, zeros128,
             agg_out,
             srcv2, dstv2, vrows2, erows2, alb2, acc_sh, sem):
        cid = lax.axis_index("c")
        sid = lax.axis_index("s")
        wid = cid * NS + sid

        pltpu.sync_copy(zeros128.at[pl.ds(sid * NPT, NPT)],
                        acc_sh.at[pl.ds(sid * NPT, NPT)])
        plsc.subcore_barrier()

        trips = (NCC + NW - 1) // NW

        def prefetch(g, par):
            @pl.when(g < NCC)
            def _():
                base = g * CC
                pltpu.sync_copy(srcI.at[pl.ds(base, CC)], srcv2[par])
                pltpu.sync_copy(dstI.at[pl.ds(base, CC)], dstv2[par])
                pltpu.async_copy(vh.at[srcv2[par]], vrows2[par], sem)
                pltpu.async_copy(eh.at[pl.ds(base, CC)], erows2[par], sem)
                pltpu.async_copy(alT.at[pl.ds(h0 * E + base, CC)],
                                 alb2[par][0], sem)
                pltpu.async_copy(alT.at[pl.ds((h0 + 1) * E + base, CC)],
                                 alb2[par][1], sem)

        def subchunk(g, par):
            srcv, dstv = srcv2[par], dstv2[par]
            vrows, erows, alb = vrows2[par], erows2[par], alb2[par]

            @pl.when(g < NCC)
            def _():
                base = g * CC
                pltpu.make_async_copy(vh.at[srcv], vrows, sem).wait()
                pltpu.make_async_copy(eh.at[pl.ds(base, CC)], erows,
                                      sem).wait()
                pltpu.make_async_copy(alT.at[pl.ds(h0 * E + base, CC)],
                                      alb[0], sem).wait()
                pltpu.make_async_copy(alT.at[pl.ds((h0 + 1) * E + base, CC)],
                                      alb[1], sem).wait()
                prefetch(g + NW, 1 - par)
                for eg in range(CC // 16):
                    sl = pl.ds(eg * 16, 16)
                    av = [alb[0][sl], alb[1][sl]]
                    for j in range(16):
                        i = eg * 16 + j
                        s0 = jnp.full((16,), av[0][j], _f32)
                        s1 = jnp.full((16,), av[1][j], _f32)
                        for c in range(8):
                            sc = pl.ds(c * 16, 16)
                            s = s0 if c < 4 else s1
                            vrows[i, sc] = (vrows[i, sc] + erows[i, sc]) * s
                pltpu.sync_copy(vrows, acc_sh.at[dstv], add=True)

        prefetch(wid, 0)
        trips2 = (trips + 1) // 2

        def chunk2(t2, carry):
            g = wid + (2 * t2) * NW
            subchunk(g, 0)
            subchunk(g + NW, 1)
            return carry

        lax.fori_loop(0, trips2, chunk2, 0)
        plsc.subcore_barrier()
        pltpu.sync_copy(acc_sh.at[pl.ds(sid * NPT, NPT)],
                        agg_out.at[pl.ds(cid * NPAD + sid * NPT, NPT)])

    return body


_sc_pass_c = (_make_pass_c(0), _make_pass_c(1))


def _rel_conv(x_src, x_dst, ei, ea, Wk, Wq, Wv, We, zeros128):
    src = ei[0]
    dst = ei[1]
    klo, khi, qlo, qhi, vlo, vhi = _qkv(x_src, x_dst, Wk, Wq, Wv)
    elo, ehi = _edense(ea, We)
    exT, s_parts = _sc_pass_a(klo, khi, qlo, qhi, elo, ehi, src, dst)
    r = _recip(s_parts)
    alT = _sc_pass_b(exT, r, dst)
    agg_lo = _sc_pass_c[0](vlo, elo, alT, src, dst, zeros128)
    agg_hi = _sc_pass_c[1](vhi, ehi, alT, src, dst, zeros128)
    return agg_lo, agg_hi


def kernel(x_user, x_item, ei_ut, ei_tu, ea_ut, ea_tu,
           Wk_ut, Wq_ut, Wv_ut, We_ut, Weo_ut,
           Wk_tu, Wq_tu, Wv_tu, We_tu, Weo_tu,
           g_user, b_user, g_item, b_item,
           ge_ut, be_ut, ge_tu, be_tu):
    zeros128 = jnp.zeros((NPAD, 128), _f32)
    ilo, ihi = _rel_conv(x_user, x_item, ei_ut, ea_ut,
                         Wk_ut, Wq_ut, Wv_ut, We_ut, zeros128)
    ulo, uhi = _rel_conv(x_item, x_user, ei_tu, ea_tu,
                         Wk_tu, Wq_tu, Wv_tu, We_tu, zeros128)
    xu = _bn_elu(ulo, uhi, x_user, g_user, b_user)
    xi = _bn_elu(ilo, ihi, x_item, g_item, b_item)
    eu = _eout_bn_elu(ea_ut, We_ut, Weo_ut, ge_ut, be_ut)
    et = _eout_bn_elu(ea_tu, We_tu, Weo_tu, ge_tu, be_tu)
    return (xu, xi, eu, et)
